# Initial kernel scaffold; baseline (speedup 1.0000x reference)
import jax
import jax.numpy as jnp
from jax.experimental import pallas as pl


def kernel(q, k, rel_emb):
    raise NotImplementedError("write your pallas kernel here")


# speedup vs baseline: 8.2335x; 8.2335x over previous
import jax
import jax.numpy as jnp
from jax.experimental import pallas as pl
from jax.experimental.pallas import tpu as pltpu

Q_LEN = 2048
K_LEN = 2048
EMB = 64
BQ = 8


def _body(rev_ref, out_ref):
    i0 = pl.program_id(0) * BQ
    for r in range(BQ):
        out_ref[r] = rev_ref[pl.ds(K_LEN - 1 - (i0 + r), K_LEN), :]


def kernel(q, k, rel_emb):
    rev = jnp.flip(rel_emb, axis=0)
    out = pl.pallas_call(
        _body,
        grid=(Q_LEN // BQ,),
        in_specs=[
            pl.BlockSpec((2 * K_LEN, EMB), lambda g: (0, 0),
                         memory_space=pltpu.VMEM),
        ],
        out_specs=pl.BlockSpec((BQ, K_LEN, EMB), lambda g: (g, 0, 0)),
        out_shape=jax.ShapeDtypeStruct((Q_LEN, K_LEN, EMB), jnp.float32),
    )(rev)
    return out
